# Initial kernel scaffold; baseline (speedup 1.0000x reference)
import jax
import jax.numpy as jnp
from jax.experimental import pallas as pl


def kernel(x, edge_index, edge_attr, Wq, bq, Wk, bk, Wv, bv, Wo, bo, w_dist, w_galign, tau):
    raise NotImplementedError("write your pallas kernel here")


# speedup vs baseline: 2.3036x; 2.3036x over previous
import functools

import jax
import jax.numpy as jnp
from jax import lax
from jax.experimental import pallas as pl
from jax.experimental.pallas import tpu as pltpu
from jax.experimental.pallas import tpu_sc as plsc

N = 10000
E = 160000
HIDDEN = 256
HALF = 128
HEADS = 4
NSUB = 16
EB = 400
EPW = E // NSUB
NBLK = EPW // EB
NPAD = 10240
RPS = NPAD // NSUB
EB2 = 80
NBLK2 = EPW // EB2
LANES = 16


def _proj_body(x_ref, wq_ref, bq_ref, wk_ref, bk_ref, wv_ref, bv_ref,
               q0_ref, q1_ref, k0_ref, k1_ref, v0_ref, v1_ref):
    xb = x_ref[...]

    def proj(w_ref, b_ref):
        y = lax.dot_general(xb, w_ref[...], (((1,), (1,)), ((), ())),
                            preferred_element_type=jnp.float32)
        return y + b_ref[...]

    q = proj(wq_ref, bq_ref)
    q0_ref[...] = q[:, :HALF]
    q1_ref[...] = q[:, HALF:]
    k = proj(wk_ref, bk_ref)
    k0_ref[...] = k[:, :HALF]
    k1_ref[...] = k[:, HALF:]
    v = proj(wv_ref, bv_ref)
    v0_ref[...] = v[:, :HALF]
    v1_ref[...] = v[:, HALF:]


def _proj(x, Wq, bq, Wk, bk, Wv, bv):
    RB = 2000
    w_spec = pl.BlockSpec((HIDDEN, HIDDEN), lambda i: (0, 0))
    b_spec = pl.BlockSpec((1, HIDDEN), lambda i: (0, 0))
    half_spec = pl.BlockSpec((RB, HALF), lambda i: (i, 0))
    return pl.pallas_call(
        _proj_body,
        grid=(N // RB,),
        in_specs=[pl.BlockSpec((RB, HIDDEN), lambda i: (i, 0)),
                  w_spec, b_spec, w_spec, b_spec, w_spec, b_spec],
        out_specs=[half_spec] * 6,
        out_shape=[jax.ShapeDtypeStruct((N, HALF), jnp.float32)] * 6,
    )(x, Wq, bq.reshape(1, HIDDEN), Wk, bk.reshape(1, HIDDEN),
      Wv, bv.reshape(1, HIDDEN))


def _logits_sc(q0, q1, k0, k1, src, tgt):
    mesh = plsc.VectorSubcoreMesh(core_axis_name="c", subcore_axis_name="s")

    @functools.partial(
        pl.kernel, mesh=mesh,
        compiler_params=pltpu.CompilerParams(needs_layout_passes=False),
        out_type=tuple(jax.ShapeDtypeStruct((E,), jnp.float32)
                       for _ in range(HEADS)),
        scratch_types=[
            pltpu.VMEM((EB,), jnp.int32),
            pltpu.VMEM((EB,), jnp.int32),
            pltpu.VMEM((EB, HALF), jnp.float32),
            pltpu.VMEM((EB, HALF), jnp.float32),
            pltpu.VMEM((EB,), jnp.float32),
            pltpu.VMEM((EB,), jnp.float32),
            pltpu.SemaphoreType.DMA,
            pltpu.SemaphoreType.DMA,
        ],
    )
    def logits_kernel(q0h, q1h, k0h, k1h, srch, tgth, o0, o1, o2, o3,
                      srcv, tgtv, qv, kv, l0v, l1v, sq, sk):
        sid = lax.axis_index("s")
        cid = lax.axis_index("c")

        def body(qh, kh, oa, ob):
            base0 = sid * EPW

            def blk(b, carry):
                base = base0 + b * EB
                pltpu.sync_copy(srch.at[pl.ds(base, EB)], srcv)
                pltpu.sync_copy(tgth.at[pl.ds(base, EB)], tgtv)
                cq = pltpu.async_copy(qh.at[tgtv], qv, sq)
                ck = pltpu.async_copy(kh.at[srcv], kv, sk)
                cq.wait()
                ck.wait()

                lane = lax.iota(jnp.int32, LANES)

                def grp(g, c2):
                    l0vec = jnp.zeros((LANES,), jnp.float32)
                    l1vec = jnp.zeros((LANES,), jnp.float32)
                    for i in range(LANES):
                        e = g * LANES + i
                        a0 = qv[e, pl.ds(0, LANES)] * kv[e, pl.ds(0, LANES)]
                        a1 = qv[e, pl.ds(64, LANES)] * kv[e, pl.ds(64, LANES)]
                        for j in range(1, 4):
                            a0 = a0 + qv[e, pl.ds(16 * j, LANES)] * kv[e, pl.ds(16 * j, LANES)]
                            a1 = a1 + qv[e, pl.ds(64 + 16 * j, LANES)] * kv[e, pl.ds(64 + 16 * j, LANES)]
                        l0vec = jnp.where(lane == i, jnp.sum(a0), l0vec)
                        l1vec = jnp.where(lane == i, jnp.sum(a1), l1vec)
                    l0v[pl.ds(g * LANES, LANES)] = l0vec
                    l1v[pl.ds(g * LANES, LANES)] = l1vec
                    return c2

                lax.fori_loop(0, EB // LANES, grp, 0)
                pltpu.sync_copy(l0v, oa.at[pl.ds(base, EB)])
                pltpu.sync_copy(l1v, ob.at[pl.ds(base, EB)])
                return carry

            lax.fori_loop(0, NBLK, blk, 0)

        @pl.when(cid == 0)
        def _():
            body(q0h, k0h, o0, o1)

        @pl.when(cid == 1)
        def _():
            body(q1h, k1h, o2, o3)

    return logits_kernel(q0, q1, k0, k1, src, tgt)


def _softmax_body(raw_ref, dg_ref, tau_ref, wd_ref, wg_ref, w_ref):
    tau_c = jnp.clip(tau_ref[0], 0.5, 5.0)
    scale = 1.0 / (8.0 * tau_c)
    logit = (raw_ref[...] * scale
             + wd_ref[0] * dg_ref[0:1, :]
             + wg_ref[0] * dg_ref[1:2, :])
    m = jnp.max(logit, axis=1, keepdims=True)
    p = jnp.exp(logit - m)
    s = jnp.sum(p, axis=1, keepdims=True)
    w_ref[...] = p / s


def _softmax(raw, dg, tau, w_dist, w_galign):
    s_spec = pl.BlockSpec(memory_space=pltpu.SMEM)
    return pl.pallas_call(
        _softmax_body,
        in_specs=[pl.BlockSpec((HEADS, E), lambda: (0, 0)),
                  pl.BlockSpec((2, E), lambda: (0, 0)),
                  s_spec, s_spec, s_spec],
        out_specs=pl.BlockSpec((HEADS, E), lambda: (0, 0)),
        out_shape=jax.ShapeDtypeStruct((HEADS, E), jnp.float32),
    )(raw, dg, tau, w_dist, w_galign)


def _scatter_sc(v0, v1, src, tgt, wa, wb, wc, wd):
    mesh = plsc.VectorSubcoreMesh(core_axis_name="c", subcore_axis_name="s")

    @functools.partial(
        pl.kernel, mesh=mesh,
        compiler_params=pltpu.CompilerParams(needs_layout_passes=False),
        out_type=(jax.ShapeDtypeStruct((NPAD, HALF), jnp.float32),
                  jax.ShapeDtypeStruct((NPAD, HALF), jnp.float32)),
        scratch_types=[
            pltpu.VMEM((EB2,), jnp.int32),
            pltpu.VMEM((EB2,), jnp.int32),
            pltpu.VMEM((EB2, HALF), jnp.float32),
            pltpu.VMEM((EB2,), jnp.float32),
            pltpu.VMEM((EB2,), jnp.float32),
            pltpu.VMEM_SHARED((NPAD, HALF), jnp.float32),
            pltpu.SemaphoreType.DMA,
        ],
    )
    def scatter_kernel(v0h, v1h, srch, tgth, wah, wbh, wch, wdh, out0, out1,
                       srcv, tgtv, vv, w0v, w1v, acc, sv):
        sid = lax.axis_index("s")
        cid = lax.axis_index("c")

        def zrow(r, c2):
            for j in range(HALF // LANES):
                vv[r, pl.ds(16 * j, LANES)] = jnp.zeros((LANES,), jnp.float32)
            return c2

        lax.fori_loop(0, EB2, zrow, 0)
        for t in range(RPS // EB2):
            pltpu.sync_copy(vv, acc.at[pl.ds(sid * RPS + t * EB2, EB2)])
        plsc.subcore_barrier()

        def scatter_phase(vh, w0h, w1h):
            base0 = sid * EPW

            def blk(b, carry):
                base = base0 + b * EB2
                pltpu.sync_copy(srch.at[pl.ds(base, EB2)], srcv)
                pltpu.sync_copy(tgth.at[pl.ds(base, EB2)], tgtv)
                cv = pltpu.async_copy(vh.at[srcv], vv, sv)
                pltpu.sync_copy(w0h.at[pl.ds(base, EB2)], w0v)
                pltpu.sync_copy(w1h.at[pl.ds(base, EB2)], w1v)
                cv.wait()

                def grp(g, c2):
                    w0vec = w0v[pl.ds(g * LANES, LANES)]
                    w1vec = w1v[pl.ds(g * LANES, LANES)]
                    for i in range(LANES):
                        e = g * LANES + i
                        w0 = w0vec[i]
                        w1 = w1vec[i]
                        for j in range(4):
                            vv[e, pl.ds(16 * j, LANES)] = vv[e, pl.ds(16 * j, LANES)] * w0
                        for j in range(4, 8):
                            vv[e, pl.ds(16 * j, LANES)] = vv[e, pl.ds(16 * j, LANES)] * w1
                    return c2

                lax.fori_loop(0, EB2 // LANES, grp, 0)
                pltpu.sync_copy(vv, acc.at[tgtv], add=True)
                return carry

            lax.fori_loop(0, NBLK2, blk, 0)

        @pl.when(cid == 0)
        def _():
            scatter_phase(v0h, wah, wbh)

        @pl.when(cid == 1)
        def _():
            scatter_phase(v1h, wch, wdh)

        plsc.subcore_barrier()

        def writeback(outh):
            r = sid * RPS
            pltpu.sync_copy(acc.at[pl.ds(r, RPS)], outh.at[pl.ds(r, RPS)])

        @pl.when(cid == 0)
        def _():
            writeback(out0)

        @pl.when(cid == 1)
        def _():
            writeback(out1)

    return scatter_kernel(v0, v1, src, tgt, wa, wb, wc, wd)


def _out_body(o0_ref, o1_ref, wo_ref, bo_ref, f_ref):
    w = wo_ref[...]
    f = lax.dot_general(o0_ref[...], w[:, :HALF], (((1,), (1,)), ((), ())),
                        preferred_element_type=jnp.float32)
    f = f + lax.dot_general(o1_ref[...], w[:, HALF:], (((1,), (1,)), ((), ())),
                            preferred_element_type=jnp.float32)
    f_ref[...] = f + bo_ref[...]


def _outproj(o0, o1, Wo, bo):
    RB = 2000
    return pl.pallas_call(
        _out_body,
        grid=(N // RB,),
        in_specs=[pl.BlockSpec((RB, HALF), lambda i: (i, 0)),
                  pl.BlockSpec((RB, HALF), lambda i: (i, 0)),
                  pl.BlockSpec((HIDDEN, HIDDEN), lambda i: (0, 0)),
                  pl.BlockSpec((1, HIDDEN), lambda i: (0, 0))],
        out_specs=pl.BlockSpec((RB, HIDDEN), lambda i: (i, 0)),
        out_shape=jax.ShapeDtypeStruct((N, HIDDEN), jnp.float32),
    )(o0, o1, Wo, bo.reshape(1, HIDDEN))


def kernel(x, edge_index, edge_attr, Wq, bq, Wk, bk, Wv, bv, Wo, bo,
           w_dist, w_galign, tau):
    src = edge_index[0].astype(jnp.int32)
    tgt = edge_index[1].astype(jnp.int32)
    dg = edge_attr[:, 2:4].T.astype(jnp.float32)

    q0, q1, k0, k1, v0, v1 = _proj(x, Wq, bq, Wk, bk, Wv, bv)
    r0, r1, r2, r3 = _logits_sc(q0, q1, k0, k1, src, tgt)
    raw = jnp.stack([r0, r1, r2, r3])
    w = _softmax(raw, dg, tau, w_dist, w_galign)
    o0p, o1p = _scatter_sc(v0, v1, src, tgt, w[0], w[1], w[2], w[3])
    return _outproj(o0p[:N], o1p[:N], Wo, bo)


# speedup vs baseline: 3.8239x; 1.6599x over previous
import functools

import jax
import jax.numpy as jnp
from jax import lax
from jax.experimental import pallas as pl
from jax.experimental.pallas import tpu as pltpu
from jax.experimental.pallas import tpu_sc as plsc

N = 10000
E = 160000
HIDDEN = 256
HALF = 128
HEADS = 4
NSUB = 16
EB = 400
EPW = E // NSUB
NBLK = EPW // EB
NPAD = 10240
RPS = NPAD // NSUB
EB2 = 80
NBLK2 = EPW // EB2
LANES = 16
EBP = 80
NBP = EPW // EBP
GRP = EBP // LANES


def _proj_body(x_ref, wq_ref, bq_ref, wk_ref, bk_ref, wv_ref, bv_ref,
               q0_ref, q1_ref, k0_ref, k1_ref, v0_ref, v1_ref):
    xb = x_ref[...]

    def proj(w_ref, b_ref):
        y = lax.dot_general(xb, w_ref[...], (((1,), (1,)), ((), ())),
                            preferred_element_type=jnp.float32)
        return y + b_ref[...]

    q = proj(wq_ref, bq_ref)
    q0_ref[...] = q[:, :HALF]
    q1_ref[...] = q[:, HALF:]
    k = proj(wk_ref, bk_ref)
    k0_ref[...] = k[:, :HALF]
    k1_ref[...] = k[:, HALF:]
    v = proj(wv_ref, bv_ref)
    v0_ref[...] = v[:, :HALF]
    v1_ref[...] = v[:, HALF:]


def _proj(x, Wq, bq, Wk, bk, Wv, bv):
    RB = 2000
    w_spec = pl.BlockSpec((HIDDEN, HIDDEN), lambda i: (0, 0))
    b_spec = pl.BlockSpec((1, HIDDEN), lambda i: (0, 0))
    half_spec = pl.BlockSpec((RB, HALF), lambda i: (i, 0))
    return pl.pallas_call(
        _proj_body,
        grid=(N // RB,),
        in_specs=[pl.BlockSpec((RB, HIDDEN), lambda i: (i, 0)),
                  w_spec, b_spec, w_spec, b_spec, w_spec, b_spec],
        out_specs=[half_spec] * 6,
        out_shape=[jax.ShapeDtypeStruct((N, HALF), jnp.float32)] * 6,
    )(x, Wq, bq.reshape(1, HIDDEN), Wk, bk.reshape(1, HIDDEN),
      Wv, bv.reshape(1, HIDDEN))


def _logits_sc(q0, q1, k0, k1, src, tgt):
    mesh = plsc.VectorSubcoreMesh(core_axis_name="c", subcore_axis_name="s")

    @functools.partial(
        pl.kernel, mesh=mesh,
        compiler_params=pltpu.CompilerParams(needs_layout_passes=False),
        out_type=tuple(jax.ShapeDtypeStruct((E,), jnp.float32)
                       for _ in range(HEADS)),
        scratch_types=[
            pltpu.VMEM((EPW,), jnp.int32),
            pltpu.VMEM((EPW,), jnp.int32),
            pltpu.VMEM((EBP, HALF), jnp.float32),
            pltpu.VMEM((EBP, HALF), jnp.float32),
            pltpu.VMEM((EBP, HALF), jnp.float32),
            pltpu.VMEM((EBP, HALF), jnp.float32),
            pltpu.VMEM((EBP,), jnp.float32),
            pltpu.VMEM((EBP,), jnp.float32),
            pltpu.SemaphoreType.DMA,
            pltpu.SemaphoreType.DMA,
        ],
    )
    def logits_kernel(q0h, q1h, k0h, k1h, srch, tgth, o0, o1, o2, o3,
                      srcv, tgtv, qA, kA, qB, kB, l0v, l1v, semA, semB):
        sid = lax.axis_index("s")
        cid = lax.axis_index("c")
        base0 = sid * EPW

        def body(qh, kh, oa, ob):
            pltpu.sync_copy(srch.at[pl.ds(base0, EPW)], srcv)
            pltpu.sync_copy(tgth.at[pl.ds(base0, EPW)], tgtv)
            lane = lax.iota(jnp.int32, LANES)

            def issue(b, qb, kb, sem):
                off = b * EBP
                pltpu.async_copy(qh.at[tgtv.at[pl.ds(off, EBP)]], qb, sem)
                pltpu.async_copy(kh.at[srcv.at[pl.ds(off, EBP)]], kb, sem)

            def drain(qb, kb, sem):
                pltpu.make_async_copy(qh.at[pl.ds(0, EBP)], qb, sem).wait()
                pltpu.make_async_copy(kh.at[pl.ds(0, EBP)], kb, sem).wait()

            def compute(b, qb, kb):
                def grp(g, c2):
                    l0vec = jnp.zeros((LANES,), jnp.float32)
                    l1vec = jnp.zeros((LANES,), jnp.float32)
                    for i in range(LANES):
                        e = g * LANES + i
                        a0 = qb[e, pl.ds(0, LANES)] * kb[e, pl.ds(0, LANES)]
                        a1 = qb[e, pl.ds(64, LANES)] * kb[e, pl.ds(64, LANES)]
                        for j in range(1, 4):
                            a0 = a0 + qb[e, pl.ds(16 * j, LANES)] * kb[e, pl.ds(16 * j, LANES)]
                            a1 = a1 + qb[e, pl.ds(64 + 16 * j, LANES)] * kb[e, pl.ds(64 + 16 * j, LANES)]
                        l0vec = jnp.where(lane == i, jnp.sum(a0), l0vec)
                        l1vec = jnp.where(lane == i, jnp.sum(a1), l1vec)
                    l0v[pl.ds(g * LANES, LANES)] = l0vec
                    l1v[pl.ds(g * LANES, LANES)] = l1vec
                    return c2

                lax.fori_loop(0, GRP, grp, 0)
                base = base0 + b * EBP
                pltpu.sync_copy(l0v, oa.at[pl.ds(base, EBP)])
                pltpu.sync_copy(l1v, ob.at[pl.ds(base, EBP)])

            issue(0, qA, kA, semA)

            def pair(t, carry):
                b0 = 2 * t
                issue(b0 + 1, qB, kB, semB)
                drain(qA, kA, semA)
                compute(b0, qA, kA)
                issue(b0 + 2, qA, kA, semA)
                drain(qB, kB, semB)
                compute(b0 + 1, qB, kB)
                return carry

            lax.fori_loop(0, (NBP - 1) // 2, pair, 0)
            drain(qA, kA, semA)
            compute(NBP - 1, qA, kA)

        @pl.when(cid == 0)
        def _():
            body(q0h, k0h, o0, o1)

        @pl.when(cid == 1)
        def _():
            body(q1h, k1h, o2, o3)

    return logits_kernel(q0, q1, k0, k1, src, tgt)


def _softmax_body(raw_ref, dg_ref, tau_ref, wd_ref, wg_ref, w_ref):
    tau_c = jnp.clip(tau_ref[0], 0.5, 5.0)
    scale = 1.0 / (8.0 * tau_c)
    logit = (raw_ref[...] * scale
             + wd_ref[0] * dg_ref[0:1, :]
             + wg_ref[0] * dg_ref[1:2, :])
    m = jnp.max(logit, axis=1, keepdims=True)
    p = jnp.exp(logit - m)
    s = jnp.sum(p, axis=1, keepdims=True)
    w_ref[...] = p / s


def _softmax(raw, dg, tau, w_dist, w_galign):
    s_spec = pl.BlockSpec(memory_space=pltpu.SMEM)
    return pl.pallas_call(
        _softmax_body,
        in_specs=[pl.BlockSpec((HEADS, E), lambda: (0, 0)),
                  pl.BlockSpec((2, E), lambda: (0, 0)),
                  s_spec, s_spec, s_spec],
        out_specs=pl.BlockSpec((HEADS, E), lambda: (0, 0)),
        out_shape=jax.ShapeDtypeStruct((HEADS, E), jnp.float32),
    )(raw, dg, tau, w_dist, w_galign)


def _scatter_sc(v0, v1, src, tgt, wa, wb, wc, wd):
    mesh = plsc.VectorSubcoreMesh(core_axis_name="c", subcore_axis_name="s")

    @functools.partial(
        pl.kernel, mesh=mesh,
        compiler_params=pltpu.CompilerParams(needs_layout_passes=False),
        out_type=(jax.ShapeDtypeStruct((NPAD, HALF), jnp.float32),
                  jax.ShapeDtypeStruct((NPAD, HALF), jnp.float32)),
        scratch_types=[
            pltpu.VMEM((EPW,), jnp.int32),
            pltpu.VMEM((EPW,), jnp.int32),
            pltpu.VMEM((EBP, HALF), jnp.float32),
            pltpu.VMEM((EBP, HALF), jnp.float32),
            pltpu.VMEM((EBP,), jnp.float32),
            pltpu.VMEM((EBP,), jnp.float32),
            pltpu.VMEM((EBP,), jnp.float32),
            pltpu.VMEM((EBP,), jnp.float32),
            pltpu.VMEM_SHARED((NPAD, HALF), jnp.float32),
            pltpu.SemaphoreType.DMA,
            pltpu.SemaphoreType.DMA,
        ],
    )
    def scatter_kernel(v0h, v1h, srch, tgth, wah, wbh, wch, wdh, out0, out1,
                       srcv, tgtv, vA, vB, w0A, w1A, w0B, w1B, acc,
                       semA, semB):
        sid = lax.axis_index("s")
        cid = lax.axis_index("c")
        base0 = sid * EPW

        def zrow(r, c2):
            for j in range(HALF // LANES):
                vA[r, pl.ds(16 * j, LANES)] = jnp.zeros((LANES,), jnp.float32)
            return c2

        lax.fori_loop(0, EBP, zrow, 0)
        for t in range(RPS // EBP):
            pltpu.sync_copy(vA, acc.at[pl.ds(sid * RPS + t * EBP, EBP)])
        plsc.subcore_barrier()

        def scatter_phase(vh, w0h, w1h):
            pltpu.sync_copy(srch.at[pl.ds(base0, EPW)], srcv)
            pltpu.sync_copy(tgth.at[pl.ds(base0, EPW)], tgtv)

            def issue(b, vb, w0b, w1b, sem):
                off = b * EBP
                base = base0 + off
                pltpu.async_copy(vh.at[srcv.at[pl.ds(off, EBP)]], vb, sem)
                pltpu.async_copy(w0h.at[pl.ds(base, EBP)], w0b, sem)
                pltpu.async_copy(w1h.at[pl.ds(base, EBP)], w1b, sem)

            def drain(vb, w0b, w1b, sem):
                pltpu.make_async_copy(vh.at[pl.ds(0, EBP)], vb, sem).wait()
                pltpu.make_async_copy(w0h.at[pl.ds(0, EBP)], w0b, sem).wait()
                pltpu.make_async_copy(w1h.at[pl.ds(0, EBP)], w1b, sem).wait()

            def compute(b, vb, w0b, w1b):
                def grp(g, c2):
                    w0vec = w0b[pl.ds(g * LANES, LANES)]
                    w1vec = w1b[pl.ds(g * LANES, LANES)]
                    for i in range(LANES):
                        e = g * LANES + i
                        w0 = w0vec[i]
                        w1 = w1vec[i]
                        for j in range(4):
                            vb[e, pl.ds(16 * j, LANES)] = vb[e, pl.ds(16 * j, LANES)] * w0
                        for j in range(4, 8):
                            vb[e, pl.ds(16 * j, LANES)] = vb[e, pl.ds(16 * j, LANES)] * w1
                    return c2

                lax.fori_loop(0, GRP, grp, 0)
                off = b * EBP
                pltpu.sync_copy(vb, acc.at[tgtv.at[pl.ds(off, EBP)]], add=True)

            issue(0, vA, w0A, w1A, semA)

            def pair(t, carry):
                b0 = 2 * t
                issue(b0 + 1, vB, w0B, w1B, semB)
                drain(vA, w0A, w1A, semA)
                compute(b0, vA, w0A, w1A)
                issue(b0 + 2, vA, w0A, w1A, semA)
                drain(vB, w0B, w1B, semB)
                compute(b0 + 1, vB, w0B, w1B)
                return carry

            lax.fori_loop(0, (NBP - 1) // 2, pair, 0)
            drain(vA, w0A, w1A, semA)
            compute(NBP - 1, vA, w0A, w1A)

        @pl.when(cid == 0)
        def _():
            scatter_phase(v0h, wah, wbh)

        @pl.when(cid == 1)
        def _():
            scatter_phase(v1h, wch, wdh)

        plsc.subcore_barrier()

        def writeback(outh):
            r = sid * RPS
            pltpu.sync_copy(acc.at[pl.ds(r, RPS)], outh.at[pl.ds(r, RPS)])

        @pl.when(cid == 0)
        def _():
            writeback(out0)

        @pl.when(cid == 1)
        def _():
            writeback(out1)

    return scatter_kernel(v0, v1, src, tgt, wa, wb, wc, wd)


def _out_body(o0_ref, o1_ref, wo_ref, bo_ref, f_ref):
    w = wo_ref[...]
    f = lax.dot_general(o0_ref[...], w[:, :HALF], (((1,), (1,)), ((), ())),
                        preferred_element_type=jnp.float32)
    f = f + lax.dot_general(o1_ref[...], w[:, HALF:], (((1,), (1,)), ((), ())),
                            preferred_element_type=jnp.float32)
    f_ref[...] = f + bo_ref[...]


def _outproj(o0, o1, Wo, bo):
    RB = 2000
    return pl.pallas_call(
        _out_body,
        grid=(N // RB,),
        in_specs=[pl.BlockSpec((RB, HALF), lambda i: (i, 0)),
                  pl.BlockSpec((RB, HALF), lambda i: (i, 0)),
                  pl.BlockSpec((HIDDEN, HIDDEN), lambda i: (0, 0)),
                  pl.BlockSpec((1, HIDDEN), lambda i: (0, 0))],
        out_specs=pl.BlockSpec((RB, HIDDEN), lambda i: (i, 0)),
        out_shape=jax.ShapeDtypeStruct((N, HIDDEN), jnp.float32),
    )(o0, o1, Wo, bo.reshape(1, HIDDEN))


def kernel(x, edge_index, edge_attr, Wq, bq, Wk, bk, Wv, bv, Wo, bo,
           w_dist, w_galign, tau):
    src = edge_index[0].astype(jnp.int32)
    tgt = edge_index[1].astype(jnp.int32)
    dg = edge_attr[:, 2:4].T.astype(jnp.float32)

    q0, q1, k0, k1, v0, v1 = _proj(x, Wq, bq, Wk, bk, Wv, bv)
    r0, r1, r2, r3 = _logits_sc(q0, q1, k0, k1, src, tgt)
    raw = jnp.stack([r0, r1, r2, r3])
    w = _softmax(raw, dg, tau, w_dist, w_galign)
    o0p, o1p = _scatter_sc(v0, v1, src, tgt, w[0], w[1], w[2], w[3])
    return _outproj(o0p[:N], o1p[:N], Wo, bo)
